# baseline (speedup 1.0000x reference)
import functools

import jax
import jax.numpy as jnp
from jax import lax
from jax.experimental import pallas as pl
from jax.experimental.pallas import tpu as pltpu
from jax.experimental.pallas import tpu_sc as plsc

_B, _S, _D = 4, 8192, 256
_N = _B * _S
_NC, _NS = 2, 16
_NW = _NC * _NS
_ROWS_PER_W = _N // _NW
_CHUNK = 128
_NCHUNK = _ROWS_PER_W // _CHUNK
_LANES = 16
_DV = _D // _LANES


def _sc_add_gather(x2d, idx, table):
    mesh = plsc.VectorSubcoreMesh(core_axis_name="c", subcore_axis_name="s")

    @functools.partial(
        pl.kernel,
        mesh=mesh,
        out_type=jax.ShapeDtypeStruct((_N, _D), jnp.float32),
        scratch_types=[
            pltpu.VMEM((_CHUNK,), jnp.int32),
            pltpu.VMEM((_CHUNK, _D), jnp.float32),
            pltpu.VMEM((_CHUNK, _D), jnp.float32),
            pltpu.SemaphoreType.DMA,
        ],
    )
    def k(x_hbm, idx_hbm, tab_hbm, out_hbm, idx_v, rows_v, x_v, sem):
        cid = lax.axis_index("c")
        sid = lax.axis_index("s")
        wid = sid * _NC + cid
        base = wid * _ROWS_PER_W

        def chunk_body(c, _):
            row0 = base + c * _CHUNK
            pltpu.sync_copy(idx_hbm.at[pl.ds(row0, _CHUNK)], idx_v)
            gather = pltpu.async_copy(tab_hbm.at[idx_v], rows_v, sem)
            pltpu.sync_copy(x_hbm.at[pl.ds(row0, _CHUNK)], x_v)
            gather.wait()

            def add_row(i, _):
                for j in range(_DV):
                    sl = pl.ds(j * _LANES, _LANES)
                    x_v[i, sl] = x_v[i, sl] + rows_v[i, sl]
                return 0

            lax.fori_loop(0, _CHUNK, add_row, 0)
            pltpu.sync_copy(x_v, out_hbm.at[pl.ds(row0, _CHUNK)])
            return 0

        lax.fori_loop(0, _NCHUNK, chunk_body, 0)

    return k(x2d, idx, table)


def kernel(x, metonic_idx, phase_map):
    x2d = x.reshape(_N, _D)
    idx = metonic_idx.reshape(_N).astype(jnp.int32)
    out = _sc_add_gather(x2d, idx, phase_map)
    return out.reshape(_B, _S, _D)


# speedup vs baseline: 1.3025x; 1.3025x over previous
import functools

import jax
import jax.numpy as jnp
from jax import lax
from jax.experimental import pallas as pl
from jax.experimental.pallas import tpu as pltpu
from jax.experimental.pallas import tpu_sc as plsc

_B, _S, _D = 4, 8192, 256
_N = _B * _S
_NC, _NS = 2, 16
_NW = _NC * _NS
_ROWS_PER_W = _N // _NW
_CHUNK = 64
_NCHUNK = _ROWS_PER_W // _CHUNK
_NSLOT = 2
_LANES = 16
_DV = _D // _LANES


def _sc_add_gather(x2d, idx3, table):
    mesh = plsc.VectorSubcoreMesh(core_axis_name="c", subcore_axis_name="s")

    scratch = [pltpu.VMEM((_NCHUNK, _CHUNK), jnp.int32)]
    for _ in range(_NSLOT):
        scratch += [
            pltpu.VMEM((_CHUNK, _D), jnp.float32),
            pltpu.VMEM((_CHUNK, _D), jnp.float32),
            pltpu.VMEM((_CHUNK, _D), jnp.float32),
            pltpu.SemaphoreType.DMA,
            pltpu.SemaphoreType.DMA,
            pltpu.SemaphoreType.DMA,
        ]

    @functools.partial(
        pl.kernel,
        mesh=mesh,
        out_type=jax.ShapeDtypeStruct((_N, _D), jnp.float32),
        scratch_types=scratch,
    )
    def k(x_hbm, idx_hbm, tab_hbm, out_hbm, idx_v, *slot_args):
        cid = lax.axis_index("c")
        sid = lax.axis_index("s")
        wid = sid * _NC + cid
        base = wid * _ROWS_PER_W

        slots = [slot_args[6 * b : 6 * b + 6] for b in range(_NSLOT)]

        pltpu.sync_copy(idx_hbm.at[wid], idx_v)

        def start_in(c):
            rows_v, x_v, _, sg, sx, _ = slots[c % _NSLOT]
            row0 = base + c * _CHUNK
            gd = pltpu.async_copy(tab_hbm.at[idx_v.at[c]], rows_v, sg)
            xd = pltpu.async_copy(x_hbm.at[pl.ds(row0, _CHUNK)], x_v, sx)
            return gd, xd

        in_descs = {}
        out_descs = {}
        for c in range(_NSLOT):
            in_descs[c] = start_in(c)

        for c in range(_NCHUNK):
            rows_v, x_v, res_v, _, _, so = slots[c % _NSLOT]
            gd, xd = in_descs.pop(c)
            gd.wait()
            xd.wait()
            if c >= _NSLOT:
                out_descs.pop(c - _NSLOT).wait()

            def add_row(i, _, x_v=x_v, rows_v=rows_v, res_v=res_v):
                for j in range(_DV):
                    sl = pl.ds(j * _LANES, _LANES)
                    res_v[i, sl] = x_v[i, sl] + rows_v[i, sl]
                return 0

            lax.fori_loop(0, _CHUNK, add_row, 0)

            row0 = base + c * _CHUNK
            out_descs[c] = pltpu.async_copy(
                res_v, out_hbm.at[pl.ds(row0, _CHUNK)], so)
            if c + _NSLOT < _NCHUNK:
                in_descs[c + _NSLOT] = start_in(c + _NSLOT)

        for c in sorted(out_descs):
            out_descs[c].wait()

    return k(x2d, idx3, table)


def kernel(x, metonic_idx, phase_map):
    x2d = x.reshape(_N, _D)
    idx3 = metonic_idx.reshape(_NW, _NCHUNK, _CHUNK).astype(jnp.int32)
    out = _sc_add_gather(x2d, idx3, phase_map)
    return out.reshape(_B, _S, _D)
